# Initial kernel scaffold; baseline (speedup 1.0000x reference)
import jax
import jax.numpy as jnp
from jax.experimental import pallas as pl


def kernel(input, s1w0, s1b0, s1w1, s1b1, s1w2, s1b2, s2w0, s2b0, s2w1, s2b1, s2w2, s2b2, gw0, gb0, gw1, gb1, gw2, gb2, l1w, l1b, l2w, l2b, l3w, l3b):
    raise NotImplementedError("write your pallas kernel here")


# speedup vs baseline: 16.6166x; 16.6166x over previous
import jax
import jax.numpy as jnp
from jax.experimental import pallas as pl
from jax.experimental.pallas import tpu as pltpu

_NEG = -1e30


def _dot(a, b):
    return jnp.dot(a, b, preferred_element_type=jnp.float32)


def _dott(a, b):
    return jax.lax.dot_general(a, b, (((0,), (0,)), ((), ())),
                               preferred_element_type=jnp.float32)


def _col(xT, i):
    e = (jax.lax.broadcasted_iota(jnp.int32, (3, 1), 0) == i).astype(jnp.float32)
    return _dott(xT, e)


def _gather_lanes(srcT, idxf, n):
    F = srcT.shape[0]
    I = idxf.shape[1]
    acc = jnp.zeros((F, I), jnp.float32)
    for c in range(n):
        src = srcT[:, c * 128:(c + 1) * 128]
        loc = idxf - c * 128
        inb = (loc >= 0) & (loc < 128)
        idxc = jnp.broadcast_to(jnp.clip(loc, 0, 127), (F, I))
        g = jnp.take_along_axis(src, idxc, axis=1)
        acc = jnp.where(inb, g, acc)
    return acc


def _gather_rows(src, idx, n):
    acc = jnp.zeros(idx.shape, jnp.float32)
    for c in range(n):
        s = src[:, c * 128:(c + 1) * 128]
        loc = idx - c * 128
        inb = (loc >= 0) & (loc < 128)
        g = jnp.take_along_axis(s, jnp.clip(loc, 0, 127), axis=1)
        acc = jnp.where(inb, g, acc)
    return acc


def _first_k(d2T, r2, L):
    N, Q = d2T.shape
    v = (d2T <= r2).astype(jnp.float32)
    carry = jnp.zeros((1, Q), jnp.float32)
    chunks = []
    for c in range(N // 128):
        sc = _dot(L, v[c * 128:(c + 1) * 128, :]) + carry
        carry = sc[127:128, :]
        chunks.append(sc)
    cnt = jnp.concatenate(chunks, axis=0)
    total = carry
    sio = jax.lax.broadcasted_iota(jnp.int32, (64, Q), 0)

    def body(k, nbr):
        s = jnp.sum((cnt <= k.astype(jnp.float32)).astype(jnp.float32),
                    axis=0, keepdims=True)
        return jnp.where(sio == k, jnp.broadcast_to(s, (64, Q)), nbr)

    nbrT = jax.lax.fori_loop(0, 64, body, jnp.zeros((64, Q), jnp.float32))
    kio = jax.lax.broadcasted_iota(jnp.int32, (64, Q), 0).astype(jnp.float32)
    mT = kio < jnp.broadcast_to(total, (64, Q))
    mf = jnp.where(mT, 1.0, 0.0)
    nc = jnp.where(mT, nbrT, 0.0).astype(jnp.int32)
    return nc, mf


def _fps_body(px, py, pz, ncand, nout):
    B = px.shape[0]
    lane_c = jax.lax.broadcasted_iota(jnp.int32, (B, ncand), 1)
    lane_o = jax.lax.broadcasted_iota(jnp.int32, (B, nout), 1)

    def body(m, carry):
        dists, last, buf = carry
        oh = lane_c == last
        xl = jnp.sum(jnp.where(oh, px, 0.0), axis=1, keepdims=True)
        yl = jnp.sum(jnp.where(oh, py, 0.0), axis=1, keepdims=True)
        zl = jnp.sum(jnp.where(oh, pz, 0.0), axis=1, keepdims=True)
        dx = px - xl
        dy = py - yl
        dz = pz - zl
        d = dx * dx + dy * dy + dz * dz
        dists = jnp.minimum(dists, d)
        mx = jnp.max(dists, axis=1, keepdims=True)
        nxt = jnp.min(jnp.where(dists == mx, lane_c, ncand),
                      axis=1, keepdims=True)
        buf = jnp.where(lane_o == m, nxt, buf)
        return dists, nxt, buf

    init = (jnp.full((B, ncand), jnp.inf, jnp.float32),
            jnp.zeros((B, 1), jnp.int32),
            jnp.zeros((B, nout), jnp.int32))
    _, _, buf = jax.lax.fori_loop(1, nout, body, init)
    return buf


def _fps_kernel(pxyz_ref, idx1_ref, idx2_ref):
    px = pxyz_ref[:, 0, :]
    py = pxyz_ref[:, 1, :]
    pz = pxyz_ref[:, 2, :]
    idx1 = _fps_body(px, py, pz, 2048, 1024)
    idx1_ref[...] = idx1
    qx = _gather_rows(px, idx1, 16)
    qy = _gather_rows(py, idx1, 16)
    qz = _gather_rows(pz, idx1, 16)
    idx2_ref[...] = _fps_body(qx, qy, qz, 1024, 256)


def _mlp3T(x, w0, b0, w1, b1, w2, b2):
    h = jax.nn.relu(_dot(w0, x) + b0)
    h = jax.nn.relu(_dot(w1, h) + b1)
    return _dot(w2, h) + b2


def _masked_max_k(msg, mflat, Q):
    msg = jnp.where(jnp.broadcast_to(mflat, msg.shape) > 0.5, msg, _NEG)
    out = msg[:, 0:Q]
    for k in range(1, 64):
        out = jnp.maximum(out, msg[:, k * Q:(k + 1) * Q])
    return jax.nn.relu(out)


def _cloud_kernel(pxyz_ref, idx1_ref, idx2_ref,
                  s1w0_ref, s1b0_ref, s1w1_ref, s1b1_ref, s1w2_ref, s1b2_ref,
                  s2w0_ref, s2b0_ref, s2w1_ref, s2b1_ref, s2w2_ref, s2b2_ref,
                  gw0_ref, gb0_ref, gw1_ref, gb1_ref, gw2_ref, gb2_ref,
                  out_ref):
    pT = pxyz_ref[0]
    i1 = idx1_ref[0]
    i2 = idx2_ref[0]
    li = jax.lax.broadcasted_iota(jnp.int32, (128, 128), 0)
    lj = jax.lax.broadcasted_iota(jnp.int32, (128, 128), 1)
    L = (li >= lj).astype(jnp.float32)

    q1T = _gather_lanes(pT, i1, 16)
    q2T = _gather_lanes(q1T, i2, 8)

    pxc = _col(pT, 0)
    pyc = _col(pT, 1)
    pzc = _col(pT, 2)

    s1w0 = s1w0_ref[...]
    s1b0 = s1b0_ref[...]
    s1w1 = s1w1_ref[...]
    s1b1 = s1b1_ref[...]
    s1w2 = s1w2_ref[...]
    s1b2 = s1b2_ref[...]

    x1_tiles = []
    for t in range(4):
        qt = q1T[:, t * 256:(t + 1) * 256]
        dx = pxc - qt[0:1, :]
        dy = pyc - qt[1:2, :]
        dz = pzc - qt[2:3, :]
        d2T = dx * dx + dy * dy + dz * dz
        nbrT, mT = _first_k(d2T, 0.1 * 0.1, L)
        idxf = jnp.reshape(nbrT, (1, 64 * 256))
        mflat = jnp.reshape(mT, (1, 64 * 256))
        pg = _gather_lanes(pT, idxf, 16)
        A = _dot(s1w0, pg)
        C = _dot(s1w0, qt)
        Ct = jnp.concatenate([C] * 64, axis=1)
        h = jax.nn.relu(A - Ct + s1b0)
        h = jax.nn.relu(_dot(s1w1, h) + s1b1)
        h = _dot(s1w2, h) + s1b2
        x1_tiles.append(_masked_max_k(h, mflat, 256))
    x1T = jnp.concatenate(x1_tiles, axis=1)

    q1xc = _col(q1T, 0)
    q1yc = _col(q1T, 1)
    q1zc = _col(q1T, 2)
    dx = q1xc - q2T[0:1, :]
    dy = q1yc - q2T[1:2, :]
    dz = q1zc - q2T[2:3, :]
    d2T = dx * dx + dy * dy + dz * dz
    nbrT, mT = _first_k(d2T, 0.2 * 0.2, L)
    idxf = jnp.reshape(nbrT, (1, 64 * 256))
    mflat = jnp.reshape(mT, (1, 64 * 256))
    x1g = _gather_lanes(x1T, idxf, 8)
    q1g = _gather_lanes(q1T, idxf, 8)
    w0x = s2w0_ref[:, 0:128]
    w0r = s2w0_ref[:, 128:131]
    A = _dot(w0x, x1g) + _dot(w0r, q1g)
    C = _dot(w0r, q2T)
    Ct = jnp.concatenate([C] * 64, axis=1)
    h = jax.nn.relu(A - Ct + s2b0_ref[...])
    h = jax.nn.relu(_dot(s2w1_ref[...], h) + s2b1_ref[...])
    h = _dot(s2w2_ref[...], h) + s2b2_ref[...]
    x2T = _masked_max_k(h, mflat, 256)

    gw0x = gw0_ref[:, 0:256]
    gw0q = gw0_ref[:, 256:259]
    h = jax.nn.relu(_dot(gw0x, x2T) + _dot(gw0q, q2T) + gb0_ref[...])
    h = jax.nn.relu(_dot(gw1_ref[...], h) + gb1_ref[...])
    h = _dot(gw2_ref[...], h) + gb2_ref[...]

    m0 = jnp.max(h[:, 0:128], axis=1, keepdims=True)
    m1 = jnp.max(h[:, 128:256], axis=1, keepdims=True)
    out_ref[0] = jnp.concatenate([m0, m1], axis=1)


def _head_kernel(xg_ref, l1w_ref, l1b_ref, l2w_ref, l2b_ref, l3w_ref, l3b_ref,
                 out_ref):
    x = xg_ref[...]
    h = jax.nn.relu(jax.lax.dot_general(
        x, l1w_ref[...], (((1,), (1,)), ((), ())),
        preferred_element_type=jnp.float32) + l1b_ref[...])
    h = jax.nn.relu(jax.lax.dot_general(
        h, l2w_ref[...], (((1,), (1,)), ((), ())),
        preferred_element_type=jnp.float32) + l2b_ref[...])
    y = jax.lax.dot_general(
        h, l3w_ref[...], (((1,), (1,)), ((), ())),
        preferred_element_type=jnp.float32) + l3b_ref[...]
    si = jax.lax.broadcasted_iota(jnp.int32, (8, 16), 0)
    lj = jax.lax.broadcasted_iota(jnp.int32, (8, 16), 1)
    P = ((lj == 2 * si) | (lj == 2 * si + 1)).astype(jnp.float32)
    y8 = 0.5 * _dot(P, y)
    ci = jax.lax.broadcasted_iota(jnp.int32, (8, 7), 1)
    rq = jnp.where(ci >= 3, y8, 0.0)
    nrm = jnp.maximum(jnp.sqrt(jnp.sum(rq * rq, axis=1, keepdims=True)), 1e-12)
    scale = jnp.where(ci < 3, jnp.ones_like(y8), jnp.broadcast_to(1.0 / nrm, (8, 7)))
    out_ref[...] = y8 * scale


def kernel(input, s1w0, s1b0, s1w1, s1b1, s1w2, s1b2,
           s2w0, s2b0, s2w1, s2b1, s2w2, s2b2,
           gw0, gb0, gw1, gb1, gw2, gb2,
           l1w, l1b, l2w, l2b, l3w, l3b):
    pxyz = jnp.transpose(input, (0, 2, 1))
    idx1, idx2 = pl.pallas_call(
        _fps_kernel,
        out_shape=(jax.ShapeDtypeStruct((8, 1024), jnp.int32),
                   jax.ShapeDtypeStruct((8, 256), jnp.int32)),
    )(pxyz)

    cb = lambda s: jnp.reshape(s, (-1, 1))
    full = lambda a: pl.BlockSpec(a.shape, lambda b: (0,) * a.ndim)
    i1r = jnp.reshape(idx1, (8, 1, 1024))
    i2r = jnp.reshape(idx2, (8, 1, 256))
    wlist = [s1w0, cb(s1b0), s1w1, cb(s1b1), s1w2, cb(s1b2),
             s2w0, cb(s2b0), s2w1, cb(s2b1), s2w2, cb(s2b2),
             gw0, cb(gb0), gw1, cb(gb1), gw2, cb(gb2)]
    pooled = pl.pallas_call(
        _cloud_kernel,
        grid=(8,),
        in_specs=[pl.BlockSpec((1, 3, 2048), lambda b: (b, 0, 0)),
                  pl.BlockSpec((1, 1, 1024), lambda b: (b, 0, 0)),
                  pl.BlockSpec((1, 1, 256), lambda b: (b, 0, 0))] +
                 [full(w) for w in wlist],
        out_specs=pl.BlockSpec((1, 1024, 2), lambda b: (b, 0, 0)),
        out_shape=jax.ShapeDtypeStruct((8, 1024, 2), jnp.float32),
        compiler_params=pltpu.CompilerParams(
            dimension_semantics=("arbitrary",)),
    )(pxyz, i1r, i2r, *wlist)

    xg = jnp.reshape(jnp.transpose(pooled, (0, 2, 1)), (16, 1024))
    rb = lambda s: jnp.reshape(s, (1, -1))
    out = pl.pallas_call(
        _head_kernel,
        out_shape=jax.ShapeDtypeStruct((8, 7), jnp.float32),
    )(xg, l1w, rb(l1b), l2w, rb(l2b), l3w, rb(l3b))
    return out


# speedup vs baseline: 21.8372x; 1.3142x over previous
import jax
import jax.numpy as jnp
from jax.experimental import pallas as pl
from jax.experimental.pallas import tpu as pltpu

_NEG = -1e30


def _dot(a, b):
    return jnp.dot(a, b, preferred_element_type=jnp.float32)


def _dott(a, b):
    return jax.lax.dot_general(a, b, (((0,), (0,)), ((), ())),
                               preferred_element_type=jnp.float32)


def _col(xT, i):
    e = (jax.lax.broadcasted_iota(jnp.int32, (3, 1), 0) == i).astype(jnp.float32)
    return _dott(xT, e)


def _gather_lanes(srcT, idxf, n):
    F = srcT.shape[0]
    I = idxf.shape[1]
    acc = jnp.zeros((F, I), jnp.float32)
    for c in range(n):
        src = srcT[:, c * 128:(c + 1) * 128]
        loc = idxf - c * 128
        inb = (loc >= 0) & (loc < 128)
        idxc = jnp.broadcast_to(jnp.clip(loc, 0, 127), (F, I))
        g = jnp.take_along_axis(src, idxc, axis=1)
        acc = jnp.where(inb, g, acc)
    return acc


def _gather_rows(src, idx, n):
    acc = jnp.zeros(idx.shape, jnp.float32)
    for c in range(n):
        s = src[:, c * 128:(c + 1) * 128]
        loc = idx - c * 128
        inb = (loc >= 0) & (loc < 128)
        g = jnp.take_along_axis(s, jnp.clip(loc, 0, 127), axis=1)
        acc = jnp.where(inb, g, acc)
    return acc


def _first_k(d2T, r2, L):
    N, Q = d2T.shape
    v = (d2T <= r2).astype(jnp.float32)
    carry = jnp.zeros((1, Q), jnp.float32)
    chunks = []
    for c in range(N // 128):
        sc = _dot(L, v[c * 128:(c + 1) * 128, :]) + carry
        carry = sc[127:128, :]
        chunks.append(sc)
    cnt = jnp.concatenate(chunks, axis=0)
    total = carry
    sio = jax.lax.broadcasted_iota(jnp.int32, (64, Q), 0)

    def body(k, nbr):
        s = jnp.sum((cnt <= k.astype(jnp.float32)).astype(jnp.float32),
                    axis=0, keepdims=True)
        return jnp.where(sio == k, jnp.broadcast_to(s, (64, Q)), nbr)

    kmax = jnp.minimum(jnp.max(total), 64.0).astype(jnp.int32)
    nbrT = jax.lax.fori_loop(0, kmax, body,
                             jnp.full((64, Q), float(N), jnp.float32))
    kio = jax.lax.broadcasted_iota(jnp.int32, (64, Q), 0).astype(jnp.float32)
    mT = kio < jnp.broadcast_to(total, (64, Q))
    mf = jnp.where(mT, 1.0, 0.0)
    nc = jnp.where(mT, nbrT, 0.0).astype(jnp.int32)
    return nc, mf


def _fps_body(px, py, pz, ncand, nout):
    B = px.shape[0]
    lane_c = jax.lax.broadcasted_iota(jnp.int32, (B, ncand), 1)
    lane_o = jax.lax.broadcasted_iota(jnp.int32, (B, nout), 1)

    def body(m, carry):
        dists, last, buf = carry
        oh = lane_c == last
        xl = jnp.sum(jnp.where(oh, px, 0.0), axis=1, keepdims=True)
        yl = jnp.sum(jnp.where(oh, py, 0.0), axis=1, keepdims=True)
        zl = jnp.sum(jnp.where(oh, pz, 0.0), axis=1, keepdims=True)
        dx = px - xl
        dy = py - yl
        dz = pz - zl
        d = dx * dx + dy * dy + dz * dz
        dists = jnp.minimum(dists, d)
        mx = jnp.max(dists, axis=1, keepdims=True)
        nxt = jnp.min(jnp.where(dists == mx, lane_c, ncand),
                      axis=1, keepdims=True)
        buf = jnp.where(lane_o == m, nxt, buf)
        return dists, nxt, buf

    init = (jnp.full((B, ncand), jnp.inf, jnp.float32),
            jnp.zeros((B, 1), jnp.int32),
            jnp.zeros((B, nout), jnp.int32))
    _, _, buf = jax.lax.fori_loop(1, nout, body, init)
    return buf


def _fps_kernel(pxyz_ref, idx1_ref, idx2_ref):
    px = pxyz_ref[:, 0, :]
    py = pxyz_ref[:, 1, :]
    pz = pxyz_ref[:, 2, :]
    idx1 = _fps_body(px, py, pz, 2048, 1024)
    idx1_ref[...] = idx1
    qx = _gather_rows(px, idx1, 16)
    qy = _gather_rows(py, idx1, 16)
    qz = _gather_rows(pz, idx1, 16)
    idx2_ref[...] = _fps_body(qx, qy, qz, 1024, 256)


def _mlp3T(x, w0, b0, w1, b1, w2, b2):
    h = jax.nn.relu(_dot(w0, x) + b0)
    h = jax.nn.relu(_dot(w1, h) + b1)
    return _dot(w2, h) + b2


def _masked_max_k(msg, mflat, Q):
    msg = jnp.where(jnp.broadcast_to(mflat, msg.shape) > 0.5, msg, _NEG)
    out = msg[:, 0:Q]
    for k in range(1, 64):
        out = jnp.maximum(out, msg[:, k * Q:(k + 1) * Q])
    return jax.nn.relu(out)


def _cloud_kernel(pxyz_ref, idx1_ref, idx2_ref,
                  s1w0_ref, s1b0_ref, s1w1_ref, s1b1_ref, s1w2_ref, s1b2_ref,
                  s2w0_ref, s2b0_ref, s2w1_ref, s2b1_ref, s2w2_ref, s2b2_ref,
                  gw0_ref, gb0_ref, gw1_ref, gb1_ref, gw2_ref, gb2_ref,
                  out_ref):
    pT = pxyz_ref[0]
    i1 = idx1_ref[0]
    i2 = idx2_ref[0]
    li = jax.lax.broadcasted_iota(jnp.int32, (128, 128), 0)
    lj = jax.lax.broadcasted_iota(jnp.int32, (128, 128), 1)
    L = (li >= lj).astype(jnp.float32)

    q1T = _gather_lanes(pT, i1, 16)
    q2T = _gather_lanes(q1T, i2, 8)

    pxc = _col(pT, 0)
    pyc = _col(pT, 1)
    pzc = _col(pT, 2)

    s1w0 = s1w0_ref[...]
    s1b0 = s1b0_ref[...]
    s1w1 = s1w1_ref[...]
    s1b1 = s1b1_ref[...]
    s1w2 = s1w2_ref[...]
    s1b2 = s1b2_ref[...]

    x1_tiles = []
    for t in range(4):
        qt = q1T[:, t * 256:(t + 1) * 256]
        dx = pxc - qt[0:1, :]
        dy = pyc - qt[1:2, :]
        dz = pzc - qt[2:3, :]
        d2T = dx * dx + dy * dy + dz * dz
        nbrT, mT = _first_k(d2T, 0.1 * 0.1, L)
        idxf = jnp.reshape(nbrT, (1, 64 * 256))
        mflat = jnp.reshape(mT, (1, 64 * 256))
        pg = _gather_lanes(pT, idxf, 16)
        A = _dot(s1w0, pg)
        C = _dot(s1w0, qt)
        Ct = jnp.concatenate([C] * 64, axis=1)
        h = jax.nn.relu(A - Ct + s1b0)
        h = jax.nn.relu(_dot(s1w1, h) + s1b1)
        h = _dot(s1w2, h) + s1b2
        x1_tiles.append(_masked_max_k(h, mflat, 256))
    x1T = jnp.concatenate(x1_tiles, axis=1)

    q1xc = _col(q1T, 0)
    q1yc = _col(q1T, 1)
    q1zc = _col(q1T, 2)
    dx = q1xc - q2T[0:1, :]
    dy = q1yc - q2T[1:2, :]
    dz = q1zc - q2T[2:3, :]
    d2T = dx * dx + dy * dy + dz * dz
    nbrT, mT = _first_k(d2T, 0.2 * 0.2, L)
    idxf = jnp.reshape(nbrT, (1, 64 * 256))
    mflat = jnp.reshape(mT, (1, 64 * 256))
    x1g = _gather_lanes(x1T, idxf, 8)
    q1g = _gather_lanes(q1T, idxf, 8)
    w0x = s2w0_ref[:, 0:128]
    w0r = s2w0_ref[:, 128:131]
    A = _dot(w0x, x1g) + _dot(w0r, q1g)
    C = _dot(w0r, q2T)
    Ct = jnp.concatenate([C] * 64, axis=1)
    h = jax.nn.relu(A - Ct + s2b0_ref[...])
    h = jax.nn.relu(_dot(s2w1_ref[...], h) + s2b1_ref[...])
    h = _dot(s2w2_ref[...], h) + s2b2_ref[...]
    x2T = _masked_max_k(h, mflat, 256)

    gw0x = gw0_ref[:, 0:256]
    gw0q = gw0_ref[:, 256:259]
    h = jax.nn.relu(_dot(gw0x, x2T) + _dot(gw0q, q2T) + gb0_ref[...])
    h = jax.nn.relu(_dot(gw1_ref[...], h) + gb1_ref[...])
    h = _dot(gw2_ref[...], h) + gb2_ref[...]

    m0 = jnp.max(h[:, 0:128], axis=1, keepdims=True)
    m1 = jnp.max(h[:, 128:256], axis=1, keepdims=True)
    out_ref[0] = jnp.concatenate([m0, m1], axis=1)


def _head_kernel(xg_ref, l1w_ref, l1b_ref, l2w_ref, l2b_ref, l3w_ref, l3b_ref,
                 out_ref):
    x = xg_ref[...]
    h = jax.nn.relu(jax.lax.dot_general(
        x, l1w_ref[...], (((1,), (1,)), ((), ())),
        preferred_element_type=jnp.float32) + l1b_ref[...])
    h = jax.nn.relu(jax.lax.dot_general(
        h, l2w_ref[...], (((1,), (1,)), ((), ())),
        preferred_element_type=jnp.float32) + l2b_ref[...])
    y = jax.lax.dot_general(
        h, l3w_ref[...], (((1,), (1,)), ((), ())),
        preferred_element_type=jnp.float32) + l3b_ref[...]
    si = jax.lax.broadcasted_iota(jnp.int32, (8, 16), 0)
    lj = jax.lax.broadcasted_iota(jnp.int32, (8, 16), 1)
    P = ((lj == 2 * si) | (lj == 2 * si + 1)).astype(jnp.float32)
    y8 = 0.5 * _dot(P, y)
    ci = jax.lax.broadcasted_iota(jnp.int32, (8, 7), 1)
    rq = jnp.where(ci >= 3, y8, 0.0)
    nrm = jnp.maximum(jnp.sqrt(jnp.sum(rq * rq, axis=1, keepdims=True)), 1e-12)
    scale = jnp.where(ci < 3, jnp.ones_like(y8), jnp.broadcast_to(1.0 / nrm, (8, 7)))
    out_ref[...] = y8 * scale


def kernel(input, s1w0, s1b0, s1w1, s1b1, s1w2, s1b2,
           s2w0, s2b0, s2w1, s2b1, s2w2, s2b2,
           gw0, gb0, gw1, gb1, gw2, gb2,
           l1w, l1b, l2w, l2b, l3w, l3b):
    pxyz = jnp.transpose(input, (0, 2, 1))
    idx1, idx2 = pl.pallas_call(
        _fps_kernel,
        out_shape=(jax.ShapeDtypeStruct((8, 1024), jnp.int32),
                   jax.ShapeDtypeStruct((8, 256), jnp.int32)),
    )(pxyz)

    cb = lambda s: jnp.reshape(s, (-1, 1))
    full = lambda a: pl.BlockSpec(a.shape, lambda b: (0,) * a.ndim)
    i1r = jnp.reshape(idx1, (8, 1, 1024))
    i2r = jnp.reshape(idx2, (8, 1, 256))
    wlist = [s1w0, cb(s1b0), s1w1, cb(s1b1), s1w2, cb(s1b2),
             s2w0, cb(s2b0), s2w1, cb(s2b1), s2w2, cb(s2b2),
             gw0, cb(gb0), gw1, cb(gb1), gw2, cb(gb2)]
    pooled = pl.pallas_call(
        _cloud_kernel,
        grid=(8,),
        in_specs=[pl.BlockSpec((1, 3, 2048), lambda b: (b, 0, 0)),
                  pl.BlockSpec((1, 1, 1024), lambda b: (b, 0, 0)),
                  pl.BlockSpec((1, 1, 256), lambda b: (b, 0, 0))] +
                 [full(w) for w in wlist],
        out_specs=pl.BlockSpec((1, 1024, 2), lambda b: (b, 0, 0)),
        out_shape=jax.ShapeDtypeStruct((8, 1024, 2), jnp.float32),
        compiler_params=pltpu.CompilerParams(
            dimension_semantics=("parallel",)),
    )(pxyz, i1r, i2r, *wlist)

    xg = jnp.reshape(jnp.transpose(pooled, (0, 2, 1)), (16, 1024))
    rb = lambda s: jnp.reshape(s, (1, -1))
    out = pl.pallas_call(
        _head_kernel,
        out_shape=jax.ShapeDtypeStruct((8, 7), jnp.float32),
    )(xg, l1w, rb(l1b), l2w, rb(l2b), l3w, rb(l3b))
    return out
